# baseline (speedup 1.0000x reference)
import functools

import jax
import jax.numpy as jnp
from jax import lax
from jax.experimental import pallas as pl
from jax.experimental.pallas import tpu as pltpu
from jax.experimental.pallas import tpu_sc as plsc

_BLK = 768
_NBLK = 13
_RMAX = _NBLK * _BLK
_KBLK = 512
_NTILES = 32


def _gelu_exact(h):
    return 0.5 * h * (1.0 + lax.erf(h * (2.0 ** -0.5)))


def _route_body(g_ref, r0_ref, r1_ref, w0_ref, be_ref, nv_ref):
    g = jax.nn.sigmoid(g_ref[...])
    n, e_num = g.shape
    iota_e = lax.broadcasted_iota(jnp.int32, (n, e_num), 1)

    m1 = jnp.max(g, axis=1, keepdims=True)
    i1 = jnp.min(jnp.where(g == m1, iota_e, e_num), axis=1, keepdims=True)
    gm = jnp.where(iota_e == i1, -jnp.inf, g)
    m2 = jnp.max(gm, axis=1, keepdims=True)
    i2 = jnp.min(jnp.where(gm == m2, iota_e, e_num), axis=1, keepdims=True)
    den = m1 + m2
    w0_ref[...] = jnp.broadcast_to(m1 / den, (n, 16))

    a = (iota_e == i1).astype(jnp.float32)
    bb = (iota_e == i2).astype(jnp.float32)
    c = a + bb

    c_bf = c.astype(jnp.bfloat16)
    strip = 256
    s_strips = []
    for blk in range(n // strip):
        r_io = lax.broadcasted_iota(jnp.int32, (strip, n), 0) + blk * strip
        c_io = lax.broadcasted_iota(jnp.int32, (strip, n), 1)
        ltb = (c_io < r_io).astype(jnp.bfloat16)
        s_strips.append(
            lax.dot_general(ltb, c_bf, (((1,), (0,)), ((), ())),
                            preferred_element_type=jnp.float32))
    s = jnp.concatenate(s_strips, axis=0)

    counts = jnp.sum(c, axis=0, keepdims=True)
    nblk_e = (counts.astype(jnp.int32) + (_BLK - 1)) // _BLK
    e_r = lax.broadcasted_iota(jnp.int32, (e_num, e_num), 0)
    e_c = lax.broadcasted_iota(jnp.int32, (e_num, e_num), 1)
    lt8_incl = (e_r <= e_c).astype(jnp.float32)
    nblk_f = nblk_e.astype(jnp.float32)
    bs_incl = lax.dot_general(nblk_f, lt8_incl, (((1,), (0,)), ((), ())),
                              preferred_element_type=jnp.float32)
    off_pad = (bs_incl - nblk_f) * float(_BLK)
    total = jnp.sum(nblk_e, axis=1, keepdims=True)
    nv_ref[...] = total

    pos0 = jnp.sum(a * s, axis=1, keepdims=True)
    pos1 = jnp.sum(bb * s, axis=1, keepdims=True)
    base0 = jnp.sum(a * off_pad, axis=1, keepdims=True)
    base1 = jnp.sum(bb * off_pad, axis=1, keepdims=True)
    r0_ref[...] = (base0 + pos0).astype(jnp.int32).reshape(n // 128, 128)
    r1_ref[...] = (base1 + pos1).astype(jnp.int32).reshape(n // 128, 128)

    j_iota = lax.broadcasted_iota(jnp.int32, (_NBLK, e_num), 0)
    jc = jnp.minimum(j_iota, total[0, 0] - 1)
    bs_i = jnp.broadcast_to(bs_incl.astype(jnp.int32), (_NBLK, e_num))
    be = jnp.sum((bs_i <= jc).astype(jnp.int32), axis=1, keepdims=True)
    be_ref[...] = be


def _route(gates):
    n, e_num = gates.shape
    return pl.pallas_call(
        _route_body,
        out_shape=[
            jax.ShapeDtypeStruct((n // 128, 128), jnp.int32),
            jax.ShapeDtypeStruct((n // 128, 128), jnp.int32),
            jax.ShapeDtypeStruct((n, 16), jnp.float32),
            jax.ShapeDtypeStruct((_NBLK, 1), jnp.int32),
            jax.ShapeDtypeStruct((1, 1), jnp.int32),
        ],
    )(gates)


def _make_dispatch(n, d):
    chunk = n // _NTILES
    mesh = plsc.VectorSubcoreMesh(core_axis_name="c", subcore_axis_name="s")

    @functools.partial(
        pl.kernel,
        mesh=mesh,
        out_type=jax.ShapeDtypeStruct((_RMAX, d), jnp.float32),
        scratch_types=[
            pltpu.VMEM((chunk,), jnp.int32),
            pltpu.VMEM((chunk,), jnp.int32),
            pltpu.VMEM((chunk, d), jnp.float32),
            pltpu.SemaphoreType.DMA,
            pltpu.SemaphoreType.DMA,
        ],
    )
    def dispatch(x_hbm, r0_hbm, r1_hbm, xs_hbm, i0_v, i1_v, x_v, s0, s1):
        wid = lax.axis_index("s") * 2 + lax.axis_index("c")
        base = wid * chunk
        pltpu.sync_copy(r0_hbm.at[pl.ds(base, chunk)], i0_v)
        pltpu.sync_copy(r1_hbm.at[pl.ds(base, chunk)], i1_v)
        pltpu.sync_copy(x_hbm.at[pl.ds(base, chunk)], x_v)
        c0 = pltpu.async_copy(x_v, xs_hbm.at[i0_v], s0)
        c1 = pltpu.async_copy(x_v, xs_hbm.at[i1_v], s1)
        c0.wait()
        c1.wait()

    return dispatch


def _make_combine(n, d):
    chunk = n // _NTILES
    mesh = plsc.VectorSubcoreMesh(core_axis_name="c", subcore_axis_name="s")

    @functools.partial(
        pl.kernel,
        mesh=mesh,
        out_type=jax.ShapeDtypeStruct((n, d), jnp.float32),
        scratch_types=[
            pltpu.VMEM((chunk,), jnp.int32),
            pltpu.VMEM((chunk,), jnp.int32),
            pltpu.VMEM((chunk, 16), jnp.float32),
            pltpu.VMEM((chunk, d), jnp.float32),
            pltpu.VMEM((chunk, d), jnp.float32),
            pltpu.SemaphoreType.DMA,
            pltpu.SemaphoreType.DMA,
        ],
    )
    def combine(ys_hbm, r0_hbm, r1_hbm, w0_hbm, out_hbm,
                i0_v, i1_v, w0_v, a_v, b_v, s0, s1):
        wid = lax.axis_index("s") * 2 + lax.axis_index("c")
        base = wid * chunk
        pltpu.sync_copy(r0_hbm.at[pl.ds(base, chunk)], i0_v)
        pltpu.sync_copy(r1_hbm.at[pl.ds(base, chunk)], i1_v)
        pltpu.sync_copy(w0_hbm.at[pl.ds(base, chunk)], w0_v)
        c0 = pltpu.async_copy(ys_hbm.at[i0_v], a_v, s0)
        c1 = pltpu.async_copy(ys_hbm.at[i1_v], b_v, s1)
        c0.wait()
        c1.wait()

        nvec = d // 16

        def tok_body(t, carry):
            s0v = w0_v[t, :]

            def col_body(cc, carry2):
                sl = pl.ds(cc * 16, 16)
                bv = b_v[t, sl]
                a_v[t, sl] = bv + s0v * (a_v[t, sl] - bv)
                return carry2
            return lax.fori_loop(0, nvec, col_body, carry, unroll=8)

        lax.fori_loop(0, chunk, tok_body, 0)
        pltpu.sync_copy(a_v, out_hbm.at[pl.ds(base, chunk)])

    return combine


def _ffn_body(be_ref, nv_ref, xs_ref, w1_ref, b1_ref, w2_ref, b2_ref,
              out_ref):
    j = pl.program_id(0)

    @pl.when(j < nv_ref[0])
    def _compute():
        xb = xs_ref[...]
        d = xs_ref.shape[1]
        acc = jnp.zeros((_BLK, d), jnp.float32)
        nk = w1_ref.shape[2] // _KBLK
        for k in range(nk):
            sl = slice(k * _KBLK, (k + 1) * _KBLK)
            w1k = w1_ref[0, :, sl]
            h = lax.dot_general(xb, w1k, (((1,), (0,)), ((), ())),
                                preferred_element_type=jnp.float32,
                                precision=lax.Precision.DEFAULT)
            h = _gelu_exact(h + b1_ref[0, 0, sl][None, :])
            w2k = w2_ref[0, sl, :]
            acc += lax.dot_general(h, w2k, (((1,), (0,)), ((), ())),
                                   preferred_element_type=jnp.float32,
                                   precision=lax.Precision.DEFAULT)
        out_ref[...] = acc + b2_ref[0, 0][None, :]


def _grouped_ffn(block_expert, nvalid, xs, W1, b1, W2, b2):
    e_num, d, dff = W1.shape
    grid_spec = pltpu.PrefetchScalarGridSpec(
        num_scalar_prefetch=2,
        grid=(_NBLK,),
        in_specs=[
            pl.BlockSpec((_BLK, d),
                         lambda j, be, nv: (jnp.minimum(j, nv[0] - 1), 0)),
            pl.BlockSpec((1, d, dff), lambda j, be, nv: (be[j], 0, 0)),
            pl.BlockSpec((1, 1, dff), lambda j, be, nv: (be[j], 0, 0)),
            pl.BlockSpec((1, dff, d), lambda j, be, nv: (be[j], 0, 0)),
            pl.BlockSpec((1, 1, d), lambda j, be, nv: (be[j], 0, 0)),
        ],
        out_specs=pl.BlockSpec((_BLK, d), lambda j, be, nv: (j, 0)),
    )
    return pl.pallas_call(
        _ffn_body,
        grid_spec=grid_spec,
        out_shape=jax.ShapeDtypeStruct((_RMAX, d), jnp.float32),
        compiler_params=pltpu.CompilerParams(
            dimension_semantics=("arbitrary",),
        ),
    )(block_expert, nvalid, xs, W1, b1.reshape(e_num, 1, dff), W2,
      b2.reshape(e_num, 1, d))


def kernel(x, centroids, W1, b1, W2, b2):
    b, s, d = x.shape
    n = b * s
    u = x.reshape(n, d)

    scores = u @ centroids.T

    r0, r1, w0r, block_expert, nvalid = _route(scores)
    r0 = r0.reshape(n)
    r1 = r1.reshape(n)
    block_expert = block_expert.reshape(_NBLK)
    nvalid = nvalid.reshape(1)

    xs = _make_dispatch(n, d)(u, r0, r1)
    ys = _grouped_ffn(block_expert, nvalid, xs, W1, b1, W2, b2)
    out = _make_combine(n, d)(ys, r0, r1, w0r)
    return out.reshape(b, s, d)


# speedup vs baseline: 1.0260x; 1.0260x over previous
import functools

import jax
import jax.numpy as jnp
from jax import lax
from jax.experimental import pallas as pl
from jax.experimental.pallas import tpu as pltpu
from jax.experimental.pallas import tpu_sc as plsc

_BLK = 768
_NBLK = 13
_RMAX = _NBLK * _BLK
_KBLK = 512
_NTILES = 32


def _gelu_exact(h):
    return 0.5 * h * (1.0 + lax.erf(h * (2.0 ** -0.5)))


def _route_body(g_ref, r0_ref, r1_ref, w0_ref, be_ref, nv_ref):
    g = jax.nn.sigmoid(g_ref[...])
    n, e_num = g.shape
    iota_e = lax.broadcasted_iota(jnp.int32, (n, e_num), 1)

    m1 = jnp.max(g, axis=1, keepdims=True)
    i1 = jnp.min(jnp.where(g == m1, iota_e, e_num), axis=1, keepdims=True)
    gm = jnp.where(iota_e == i1, -jnp.inf, g)
    m2 = jnp.max(gm, axis=1, keepdims=True)
    i2 = jnp.min(jnp.where(gm == m2, iota_e, e_num), axis=1, keepdims=True)
    den = m1 + m2
    w0_ref[...] = jnp.broadcast_to(m1 / den, (n, 16))

    a = (iota_e == i1).astype(jnp.float32)
    bb = (iota_e == i2).astype(jnp.float32)
    c = a + bb

    c_bf = c.astype(jnp.bfloat16)
    strip = 256
    s_strips = []
    for blk in range(n // strip):
        r_io = lax.broadcasted_iota(jnp.int32, (strip, n), 0) + blk * strip
        c_io = lax.broadcasted_iota(jnp.int32, (strip, n), 1)
        ltb = (c_io < r_io).astype(jnp.bfloat16)
        s_strips.append(
            lax.dot_general(ltb, c_bf, (((1,), (0,)), ((), ())),
                            preferred_element_type=jnp.float32))
    s = jnp.concatenate(s_strips, axis=0)

    counts = jnp.sum(c, axis=0, keepdims=True)
    nblk_e = (counts.astype(jnp.int32) + (_BLK - 1)) // _BLK
    e_r = lax.broadcasted_iota(jnp.int32, (e_num, e_num), 0)
    e_c = lax.broadcasted_iota(jnp.int32, (e_num, e_num), 1)
    lt8_incl = (e_r <= e_c).astype(jnp.float32)
    nblk_f = nblk_e.astype(jnp.float32)
    bs_incl = lax.dot_general(nblk_f, lt8_incl, (((1,), (0,)), ((), ())),
                              preferred_element_type=jnp.float32)
    off_pad = (bs_incl - nblk_f) * float(_BLK)
    total = jnp.sum(nblk_e, axis=1, keepdims=True)
    nv_ref[...] = total

    pos0 = jnp.sum(a * s, axis=1, keepdims=True)
    pos1 = jnp.sum(bb * s, axis=1, keepdims=True)
    base0 = jnp.sum(a * off_pad, axis=1, keepdims=True)
    base1 = jnp.sum(bb * off_pad, axis=1, keepdims=True)
    r0_ref[...] = (base0 + pos0).astype(jnp.int32).reshape(n // 128, 128)
    r1_ref[...] = (base1 + pos1).astype(jnp.int32).reshape(n // 128, 128)

    j_iota = lax.broadcasted_iota(jnp.int32, (_NBLK, e_num), 0)
    jc = jnp.minimum(j_iota, total[0, 0] - 1)
    bs_i = jnp.broadcast_to(bs_incl.astype(jnp.int32), (_NBLK, e_num))
    be = jnp.sum((bs_i <= jc).astype(jnp.int32), axis=1, keepdims=True)
    be_ref[...] = be


def _route(gates):
    n, e_num = gates.shape
    return pl.pallas_call(
        _route_body,
        out_shape=[
            jax.ShapeDtypeStruct((n // 128, 128), jnp.int32),
            jax.ShapeDtypeStruct((n // 128, 128), jnp.int32),
            jax.ShapeDtypeStruct((n, 16), jnp.float32),
            jax.ShapeDtypeStruct((_NBLK, 1), jnp.int32),
            jax.ShapeDtypeStruct((1, 1), jnp.int32),
        ],
    )(gates)


def _make_dispatch(n, d):
    chunk = n // _NTILES
    mesh = plsc.VectorSubcoreMesh(core_axis_name="c", subcore_axis_name="s")

    @functools.partial(
        pl.kernel,
        mesh=mesh,
        out_type=jax.ShapeDtypeStruct((_RMAX, d), jnp.float32),
        scratch_types=[
            pltpu.VMEM((chunk,), jnp.int32),
            pltpu.VMEM((chunk,), jnp.int32),
            pltpu.VMEM((chunk, d), jnp.float32),
            pltpu.SemaphoreType.DMA,
            pltpu.SemaphoreType.DMA,
        ],
    )
    def dispatch(x_hbm, r0_hbm, r1_hbm, xs_hbm, i0_v, i1_v, x_v, s0, s1):
        wid = lax.axis_index("s") * 2 + lax.axis_index("c")
        base = wid * chunk
        pltpu.sync_copy(r0_hbm.at[pl.ds(base, chunk)], i0_v)
        pltpu.sync_copy(r1_hbm.at[pl.ds(base, chunk)], i1_v)
        pltpu.sync_copy(x_hbm.at[pl.ds(base, chunk)], x_v)
        c0 = pltpu.async_copy(x_v, xs_hbm.at[i0_v], s0)
        c1 = pltpu.async_copy(x_v, xs_hbm.at[i1_v], s1)
        c0.wait()
        c1.wait()

    return dispatch


def _make_combine(n, d):
    chunk = n // _NTILES
    mesh = plsc.VectorSubcoreMesh(core_axis_name="c", subcore_axis_name="s")

    @functools.partial(
        pl.kernel,
        mesh=mesh,
        out_type=jax.ShapeDtypeStruct((n, d), jnp.float32),
        scratch_types=[
            pltpu.VMEM((chunk,), jnp.int32),
            pltpu.VMEM((chunk,), jnp.int32),
            pltpu.VMEM((chunk, 16), jnp.float32),
            pltpu.VMEM((chunk, d), jnp.float32),
            pltpu.VMEM((chunk, d), jnp.float32),
            pltpu.SemaphoreType.DMA,
            pltpu.SemaphoreType.DMA,
        ],
    )
    def combine(ys_hbm, r0_hbm, r1_hbm, w0_hbm, out_hbm,
                i0_v, i1_v, w0_v, a_v, b_v, s0, s1):
        wid = lax.axis_index("s") * 2 + lax.axis_index("c")
        base = wid * chunk
        pltpu.sync_copy(r0_hbm.at[pl.ds(base, chunk)], i0_v)
        pltpu.sync_copy(r1_hbm.at[pl.ds(base, chunk)], i1_v)
        pltpu.sync_copy(w0_hbm.at[pl.ds(base, chunk)], w0_v)
        c0 = pltpu.async_copy(ys_hbm.at[i0_v], a_v, s0)
        c1 = pltpu.async_copy(ys_hbm.at[i1_v], b_v, s1)
        c0.wait()
        c1.wait()

        nvec = d // 16

        def tok_body(t, carry):
            s0v = w0_v[t, :]
            s1v = 1.0 - s0v

            def col_body(cc, carry2):
                sl = pl.ds(cc * 16, 16)
                a_v[t, sl] = s0v * a_v[t, sl] + s1v * b_v[t, sl]
                return carry2
            return lax.fori_loop(0, nvec, col_body, carry, unroll=8)

        lax.fori_loop(0, chunk, tok_body, 0)
        pltpu.sync_copy(a_v, out_hbm.at[pl.ds(base, chunk)])

    return combine


def _ffn_body(be_ref, nv_ref, xs_ref, w1_ref, b1_ref, w2_ref, b2_ref,
              out_ref):
    j = pl.program_id(0)

    @pl.when(j < nv_ref[0])
    def _compute():
        xb = xs_ref[...]
        d = xs_ref.shape[1]
        acc = jnp.zeros((_BLK, d), jnp.float32)
        nk = w1_ref.shape[2] // _KBLK
        for k in range(nk):
            sl = slice(k * _KBLK, (k + 1) * _KBLK)
            w1k = w1_ref[0, :, sl]
            h = lax.dot_general(xb, w1k, (((1,), (0,)), ((), ())),
                                preferred_element_type=jnp.float32,
                                precision=lax.Precision.DEFAULT)
            h = _gelu_exact(h + b1_ref[0, 0, sl][None, :])
            w2k = w2_ref[0, sl, :]
            acc += lax.dot_general(h, w2k, (((1,), (0,)), ((), ())),
                                   preferred_element_type=jnp.float32,
                                   precision=lax.Precision.DEFAULT)
        out_ref[...] = acc + b2_ref[0, 0][None, :]


def _grouped_ffn(block_expert, nvalid, xs, W1, b1, W2, b2):
    e_num, d, dff = W1.shape
    grid_spec = pltpu.PrefetchScalarGridSpec(
        num_scalar_prefetch=2,
        grid=(_NBLK,),
        in_specs=[
            pl.BlockSpec((_BLK, d),
                         lambda j, be, nv: (jnp.minimum(j, nv[0] - 1), 0)),
            pl.BlockSpec((1, d, dff), lambda j, be, nv: (be[j], 0, 0)),
            pl.BlockSpec((1, 1, dff), lambda j, be, nv: (be[j], 0, 0)),
            pl.BlockSpec((1, dff, d), lambda j, be, nv: (be[j], 0, 0)),
            pl.BlockSpec((1, 1, d), lambda j, be, nv: (be[j], 0, 0)),
        ],
        out_specs=pl.BlockSpec((_BLK, d), lambda j, be, nv: (j, 0)),
    )
    return pl.pallas_call(
        _ffn_body,
        grid_spec=grid_spec,
        out_shape=jax.ShapeDtypeStruct((_RMAX, d), jnp.float32),
        compiler_params=pltpu.CompilerParams(
            dimension_semantics=("arbitrary",),
        ),
    )(block_expert, nvalid, xs, W1, b1.reshape(e_num, 1, dff), W2,
      b2.reshape(e_num, 1, d))


def kernel(x, centroids, W1, b1, W2, b2):
    b, s, d = x.shape
    n = b * s
    u = x.reshape(n, d)

    scores = u @ centroids.T

    r0, r1, w0r, block_expert, nvalid = _route(scores)
    r0 = r0.reshape(n)
    r1 = r1.reshape(n)
    block_expert = block_expert.reshape(_NBLK)
    nvalid = nvalid.reshape(1)

    xs = _make_dispatch(n, d)(u, r0, r1)
    ys = _grouped_ffn(block_expert, nvalid, xs, W1, b1, W2, b2)
    out = _make_combine(n, d)(ys, r0, r1, w0r)
    return out.reshape(b, s, d)
